# Initial kernel scaffold; baseline (speedup 1.0000x reference)
import jax
import jax.numpy as jnp
from jax.experimental import pallas as pl


def kernel(ctx, itm, fake_y, pos, ctx_v, embed_weight, l1_weight, l1_bias):
    raise NotImplementedError("write your pallas kernel here")


# speedup vs baseline: 7.0087x; 7.0087x over previous
import functools

import jax
import jax.numpy as jnp
from jax import lax
from jax.experimental import pallas as pl
from jax.experimental.pallas import tpu as pltpu
from jax.experimental.pallas import tpu_sc as plsc

B = 16384
L = 200
D = 16
NC = 2
NS = 16
NW = NC * NS
SPW = B // NW
NB = 16
NCHUNK = SPW // NB
RPC = NB * L
GW = 128
G = RPC // GW
KFULL = L // D
LTAIL = L - KFULL * D


def _build():
  mesh = plsc.VectorSubcoreMesh(core_axis_name="c", subcore_axis_name="s")

  @functools.partial(
      pl.kernel,
      mesh=mesh,
      out_type=jax.ShapeDtypeStruct((B,), jnp.float32),
      compiler_params=pltpu.CompilerParams(needs_layout_passes=False,
                                           use_tc_tiling_on_sc=False),
      scratch_types=[
          pltpu.VMEM((RPC,), jnp.int32),
          pltpu.VMEM((RPC + D,), jnp.float32),
          pltpu.VMEM((RPC, D), jnp.float32),
          pltpu.VMEM((NB,), jnp.int32),
          pltpu.VMEM((NB, D), jnp.float32),
          pltpu.VMEM((NB,), jnp.float32),
          pltpu.VMEM((NB, D), jnp.float32),
          pltpu.VMEM((NB,), jnp.float32),
          pltpu.VMEM((D,), jnp.float32),
          pltpu.VMEM((D,), jnp.float32),
          pltpu.SemaphoreType.DMA,
          pltpu.SemaphoreType.DMA,
      ],
  )
  def disc_kernel(ctx_hbm, ctxv_hbm, itm_hbm, fy_hbm, table_hbm, l1v_hbm,
                  misc_hbm, out_hbm,
                  idx_v, ctxv_v, rows_v, itm_idx_v, itm_rows_v, fy_v,
                  prod_v, out_v, l1_v, misc_v, sem, sem_itm):
    wid = lax.axis_index("c") * NS + lax.axis_index("s")
    pltpu.sync_copy(l1v_hbm, l1_v)
    pltpu.sync_copy(misc_hbm, misc_v)

    def chunk_body(c, carry):
      sbase = wid * SPW + c * NB

      pltpu.sync_copy(ctx_hbm.at[pl.ds(sbase * L, RPC)], idx_v)
      pltpu.sync_copy(ctxv_hbm.at[pl.ds(sbase * L, RPC)],
                      ctxv_v.at[pl.ds(0, RPC)])
      pltpu.sync_copy(itm_hbm.at[pl.ds(sbase, NB)], itm_idx_v)
      pltpu.sync_copy(fy_hbm.at[pl.ds(sbase, NB)], fy_v)

      def fire(g, cc):
        pltpu.async_copy(table_hbm.at[idx_v.at[pl.ds(g * GW, GW)]],
                         rows_v.at[pl.ds(g * GW, GW)], sem)
        return cc
      lax.fori_loop(0, G, fire, 0)
      pltpu.async_copy(table_hbm.at[itm_idx_v], itm_rows_v, sem_itm).wait()

      def drain(g, cc):
        pltpu.make_async_copy(table_hbm.at[idx_v.at[pl.ds(g * GW, GW)]],
                              rows_v.at[pl.ds(g * GW, GW)], sem).wait()
        return cc
      lax.fori_loop(0, G, drain, 0)

      misc = misc_v[...]
      l1w = l1_v[...]
      fy = fy_v[...]
      lane = lax.iota(jnp.int32, D)

      for b in range(NB):
        bbase = b * L

        def k_body(k, acc, _bbase=bbase):
          off = _bbase + k * D
          w16 = ctxv_v[pl.ds(off, D)]
          for j in range(D):
            acc = acc + rows_v[off + j] * w16[j]
          return acc

        acc = lax.fori_loop(0, KFULL, k_body,
                            jnp.zeros((D,), jnp.float32))
        toff = bbase + KFULL * D
        wt = ctxv_v[pl.ds(toff, D)]
        for j in range(LTAIL):
          acc = acc + rows_v[toff + j] * wt[j]

        prod_v[b] = acc * (itm_rows_v[b] * l1w)

      colsum = jnp.zeros((D,), jnp.float32)
      for d in range(D):
        col = plsc.load_gather(prod_v, [lane, jnp.full((D,), d, jnp.int32)])
        colsum = colsum + col

      out_v[...] = colsum + fy * misc[0] + misc[1]
      pltpu.sync_copy(out_v, out_hbm.at[pl.ds(sbase, NB)])
      return carry

    lax.fori_loop(0, NCHUNK, chunk_body, 0)

  return disc_kernel


_DISC = _build()


@jax.jit
def _run(ctx_flat, ctxv_flat, itm_flat, fake_y, embed_weight, l1v, misc):
  return _DISC(ctx_flat, ctxv_flat, itm_flat, fake_y, embed_weight, l1v, misc)


def kernel(ctx, itm, fake_y, pos, ctx_v, embed_weight, l1_weight, l1_bias):
  del pos
  ctx_flat = ctx.astype(jnp.int32).reshape(B * L)
  itm_flat = itm.astype(jnp.int32).reshape(B)
  ctxv_flat = ctx_v.reshape(B * L)
  l1v = l1_weight[0, :D].astype(jnp.float32)
  misc = jnp.zeros((D,), jnp.float32)
  misc = misc.at[0].set(l1_weight[0, D]).at[1].set(l1_bias[0])
  return _run(ctx_flat, ctxv_flat, itm_flat, fake_y.astype(jnp.float32),
              embed_weight.astype(jnp.float32), l1v, misc)


# speedup vs baseline: 8.0829x; 1.1533x over previous
import functools

import jax
import jax.numpy as jnp
from jax import lax
from jax.experimental import pallas as pl
from jax.experimental.pallas import tpu as pltpu
from jax.experimental.pallas import tpu_sc as plsc

B = 16384
L = 200
D = 16
NC = 2
NS = 16
NW = NC * NS
SPW = B // NW
NB = 16
NCHUNK = SPW // NB
RPC = NB * L
GW = 128
G = RPC // GW
KFULL = L // D
LTAIL = L - KFULL * D


def _build():
  mesh = plsc.VectorSubcoreMesh(core_axis_name="c", subcore_axis_name="s")

  buf_types = [
      pltpu.VMEM((RPC,), jnp.int32),
      pltpu.VMEM((RPC + D,), jnp.float32),
      pltpu.VMEM((RPC, D), jnp.float32),
      pltpu.VMEM((NB,), jnp.int32),
      pltpu.VMEM((NB, D), jnp.float32),
      pltpu.VMEM((NB,), jnp.float32),
      pltpu.SemaphoreType.DMA,
      pltpu.SemaphoreType.DMA,
  ]

  @functools.partial(
      pl.kernel,
      mesh=mesh,
      out_type=jax.ShapeDtypeStruct((B,), jnp.float32),
      compiler_params=pltpu.CompilerParams(needs_layout_passes=False,
                                           use_tc_tiling_on_sc=False),
      scratch_types=buf_types + buf_types + [
          pltpu.VMEM((NB, D), jnp.float32),
          pltpu.VMEM((NB,), jnp.float32),
          pltpu.VMEM((D,), jnp.float32),
          pltpu.VMEM((D,), jnp.float32),
      ],
  )
  def disc_kernel(ctx_hbm, ctxv_hbm, itm_hbm, fy_hbm, table_hbm, l1v_hbm,
                  misc_hbm, out_hbm, *scratch):
    buf0 = scratch[0:8]
    buf1 = scratch[8:16]
    prod_v, out_v, l1_v, misc_v = scratch[16:20]
    wid = lax.axis_index("c") * NS + lax.axis_index("s")
    pltpu.sync_copy(l1v_hbm, l1_v)
    pltpu.sync_copy(misc_hbm, misc_v)

    def fire(c, buf):
      idx_v, ctxv_v, rows_v, itm_idx_v, itm_rows_v, fy_v, sem, sem_itm = buf
      sbase = wid * SPW + c * NB
      pltpu.sync_copy(ctx_hbm.at[pl.ds(sbase * L, RPC)], idx_v)
      pltpu.sync_copy(ctxv_hbm.at[pl.ds(sbase * L, RPC)],
                      ctxv_v.at[pl.ds(0, RPC)])
      pltpu.sync_copy(itm_hbm.at[pl.ds(sbase, NB)], itm_idx_v)
      pltpu.sync_copy(fy_hbm.at[pl.ds(sbase, NB)], fy_v)

      def fire_g(g, cc):
        pltpu.async_copy(table_hbm.at[idx_v.at[pl.ds(g * GW, GW)]],
                         rows_v.at[pl.ds(g * GW, GW)], sem)
        return cc
      lax.fori_loop(0, G, fire_g, 0)
      pltpu.async_copy(table_hbm.at[itm_idx_v], itm_rows_v, sem_itm)

    def drain(buf):
      idx_v, ctxv_v, rows_v, itm_idx_v, itm_rows_v, fy_v, sem, sem_itm = buf

      def drain_g(g, cc):
        pltpu.make_async_copy(table_hbm.at[idx_v.at[pl.ds(g * GW, GW)]],
                              rows_v.at[pl.ds(g * GW, GW)], sem).wait()
        return cc
      lax.fori_loop(0, G, drain_g, 0)
      pltpu.make_async_copy(table_hbm.at[itm_idx_v], itm_rows_v,
                            sem_itm).wait()

    def compute(c, buf):
      idx_v, ctxv_v, rows_v, itm_idx_v, itm_rows_v, fy_v, sem, sem_itm = buf
      sbase = wid * SPW + c * NB
      misc = misc_v[...]
      l1w = l1_v[...]
      fy = fy_v[...]
      lane = lax.iota(jnp.int32, D)

      for b in range(NB):
        bbase = b * L

        def k_body(k, acc, _bbase=bbase):
          off = _bbase + k * D
          w16 = ctxv_v[pl.ds(off, D)]
          for j in range(D):
            acc = acc + rows_v[off + j] * w16[j]
          return acc

        acc = lax.fori_loop(0, KFULL, k_body,
                            jnp.zeros((D,), jnp.float32))
        toff = bbase + KFULL * D
        wt = ctxv_v[pl.ds(toff, D)]
        for j in range(LTAIL):
          acc = acc + rows_v[toff + j] * wt[j]

        prod_v[b] = acc * (itm_rows_v[b] * l1w)

      colsum = jnp.zeros((D,), jnp.float32)
      for d in range(D):
        col = plsc.load_gather(prod_v, [lane, jnp.full((D,), d, jnp.int32)])
        colsum = colsum + col

      out_v[...] = colsum + fy * misc[0] + misc[1]
      pltpu.sync_copy(out_v, out_hbm.at[pl.ds(sbase, NB)])

    fire(0, buf0)

    def pair_body(c2, carry):
      c = c2 * 2
      drain(buf0)
      fire(c + 1, buf1)
      compute(c, buf0)
      drain(buf1)
      fire(lax.rem(c + 2, NCHUNK), buf0)
      compute(c + 1, buf1)
      return carry

    lax.fori_loop(0, NCHUNK // 2, pair_body, 0)
    drain(buf0)

  return disc_kernel


_DISC = _build()


@jax.jit
def _run(ctx_flat, ctxv_flat, itm_flat, fake_y, embed_weight, l1v, misc):
  return _DISC(ctx_flat, ctxv_flat, itm_flat, fake_y, embed_weight, l1v, misc)


def kernel(ctx, itm, fake_y, pos, ctx_v, embed_weight, l1_weight, l1_bias):
  del pos
  ctx_flat = ctx.astype(jnp.int32).reshape(B * L)
  itm_flat = itm.astype(jnp.int32).reshape(B)
  ctxv_flat = ctx_v.reshape(B * L)
  l1v = l1_weight[0, :D].astype(jnp.float32)
  misc = jnp.zeros((D,), jnp.float32)
  misc = misc.at[0].set(l1_weight[0, D]).at[1].set(l1_bias[0])
  return _run(ctx_flat, ctxv_flat, itm_flat, fake_y.astype(jnp.float32),
              embed_weight.astype(jnp.float32), l1v, misc)


# speedup vs baseline: 8.2924x; 1.0259x over previous
import functools

import jax
import jax.numpy as jnp
from jax import lax
from jax.experimental import pallas as pl
from jax.experimental.pallas import tpu as pltpu
from jax.experimental.pallas import tpu_sc as plsc

B = 16384
L = 200
D = 16
V = 1000000
NC = 2
NS = 16
NW = NC * NS
SPW = B // NW
NB = 16
NCHUNK = SPW // NB
RPC = NB * L
GW = 128
G = RPC // GW
KFULL = L // D
LTAIL = L - KFULL * D


def _build():
  mesh = plsc.VectorSubcoreMesh(core_axis_name="c", subcore_axis_name="s")

  buf_types = [
      pltpu.VMEM((RPC,), jnp.int32),
      pltpu.VMEM((RPC + D,), jnp.float32),
      pltpu.VMEM((RPC, D), jnp.float32),
      pltpu.VMEM((NB,), jnp.int32),
      pltpu.VMEM((NB, D), jnp.float32),
      pltpu.VMEM((NB,), jnp.float32),
      pltpu.SemaphoreType.DMA,
      pltpu.SemaphoreType.DMA,
  ]

  @functools.partial(
      pl.kernel,
      mesh=mesh,
      out_type=jax.ShapeDtypeStruct((B,), jnp.float32),
      compiler_params=pltpu.CompilerParams(needs_layout_passes=False,
                                           use_tc_tiling_on_sc=False),
      scratch_types=buf_types + buf_types + [
          pltpu.VMEM((NB, D), jnp.float32),
          pltpu.VMEM((NB,), jnp.float32),
          pltpu.VMEM((D,), jnp.float32),
          pltpu.VMEM((D,), jnp.float32),
      ],
  )
  def disc_kernel(ctx_hbm, ctxv_hbm, itm_hbm, fy_hbm, table_hbm, l1v_hbm,
                  misc_hbm, out_hbm, *scratch):
    buf0 = scratch[0:8]
    buf1 = scratch[8:16]
    prod_v, out_v, l1_v, misc_v = scratch[16:20]
    wid = lax.axis_index("c") * NS + lax.axis_index("s")
    pltpu.sync_copy(l1v_hbm, l1_v)
    pltpu.sync_copy(misc_hbm, misc_v)

    def fire(c, buf):
      idx_v, ctxv_v, rows_v, itm_idx_v, itm_rows_v, fy_v, sem, sem_itm = buf
      sbase = wid * SPW + c * NB
      pltpu.sync_copy(ctx_hbm.at[pl.ds(sbase * L, RPC)], idx_v)
      pltpu.sync_copy(ctxv_hbm.at[pl.ds(sbase * L, RPC)],
                      ctxv_v.at[pl.ds(0, RPC)])
      pltpu.sync_copy(itm_hbm.at[pl.ds(sbase, NB)], itm_idx_v)
      pltpu.sync_copy(fy_hbm.at[pl.ds(sbase, NB)], fy_v)

      def fire_g(g, cc):
        pltpu.async_copy(table_hbm.at[idx_v.at[pl.ds(g * GW, GW)]],
                         rows_v.at[pl.ds(g * GW, GW)], sem)
        return cc
      lax.fori_loop(0, G, fire_g, 0)
      pltpu.async_copy(table_hbm.at[itm_idx_v], itm_rows_v, sem_itm)

    def drain(buf):
      idx_v, ctxv_v, rows_v, itm_idx_v, itm_rows_v, fy_v, sem, sem_itm = buf

      def drain_g(g, cc):
        pltpu.make_async_copy(table_hbm.at[idx_v.at[pl.ds(g * GW, GW)]],
                              rows_v.at[pl.ds(g * GW, GW)], sem).wait()
        return cc
      lax.fori_loop(0, G, drain_g, 0)
      pltpu.make_async_copy(table_hbm.at[itm_idx_v], itm_rows_v,
                            sem_itm).wait()

    def compute(c, buf):
      idx_v, ctxv_v, rows_v, itm_idx_v, itm_rows_v, fy_v, sem, sem_itm = buf
      sbase = wid * SPW + c * NB
      misc = misc_v[...]
      l1w = l1_v[...]
      fy = fy_v[...]
      lane = lax.iota(jnp.int32, D)

      zero = jnp.zeros((D,), jnp.float32)
      for b in range(NB):
        bbase = b * L

        def k_body(k, accs, _bbase=bbase):
          off = _bbase + k * D
          w16 = ctxv_v[pl.ds(off, D)]
          accs = list(accs)
          for j in range(D):
            accs[j % 4] = accs[j % 4] + rows_v[off + j] * w16[j]
          return tuple(accs)

        accs = lax.fori_loop(0, KFULL, k_body, (zero, zero, zero, zero))
        a0, a1, a2, a3 = accs
        toff = bbase + KFULL * D
        wt = ctxv_v[pl.ds(toff, D)]
        for j in range(LTAIL):
          if j % 4 == 0:
            a0 = a0 + rows_v[toff + j] * wt[j]
          elif j % 4 == 1:
            a1 = a1 + rows_v[toff + j] * wt[j]
          elif j % 4 == 2:
            a2 = a2 + rows_v[toff + j] * wt[j]
          else:
            a3 = a3 + rows_v[toff + j] * wt[j]
        acc = (a0 + a1) + (a2 + a3)
        prod_v[b] = acc * (itm_rows_v[b] * l1w)

      colsum = jnp.zeros((D,), jnp.float32)
      for d in range(D):
        col = plsc.load_gather(prod_v, [lane, jnp.full((D,), d, jnp.int32)])
        colsum = colsum + col

      out_v[...] = colsum + fy * misc[0] + misc[1]
      pltpu.sync_copy(out_v, out_hbm.at[pl.ds(sbase, NB)])

    fire(0, buf0)

    def pair_body(c2, carry):
      c = c2 * 2
      drain(buf0)
      fire(c + 1, buf1)
      compute(c, buf0)
      drain(buf1)
      fire(lax.rem(c + 2, NCHUNK), buf0)
      compute(c + 1, buf1)
      return carry

    lax.fori_loop(0, NCHUNK // 2, pair_body, 0)
    drain(buf0)

  return disc_kernel


_DISC = _build()


@jax.jit
def _run(ctx_flat, ctxv_flat, itm_flat, fake_y, embed_weight, l1v, misc):
  return _DISC(ctx_flat, ctxv_flat, itm_flat, fake_y, embed_weight, l1v, misc)


def kernel(ctx, itm, fake_y, pos, ctx_v, embed_weight, l1_weight, l1_bias):
  del pos
  ctx_flat = ctx.astype(jnp.int32).reshape(B * L)
  table_lin = embed_weight.astype(jnp.float32)
  itm_flat = itm.astype(jnp.int32).reshape(B)
  ctxv_flat = ctx_v.reshape(B * L)
  l1v = l1_weight[0, :D].astype(jnp.float32)
  misc = jnp.zeros((D,), jnp.float32)
  misc = misc.at[0].set(l1_weight[0, D]).at[1].set(l1_bias[0])
  return _run(ctx_flat, ctxv_flat, itm_flat, fake_y.astype(jnp.float32),
              table_lin, l1v, misc)
